# baseline (speedup 1.0000x reference)
import functools

import jax
import jax.numpy as jnp
from jax import lax
from jax.experimental import pallas as pl
from jax.experimental.pallas import tpu as pltpu
from jax.experimental.pallas import tpu_sc as plsc

VOCAB = 100000
D = 64
B = 4096
HIST = 50
HP = 56
VW = 64
NW = 32
BPW = B // NW
CB = 2
NCHUNK = BPW // CB
GSZ = CB * HP
NBUF = 4
KD = D // 16

_DNUMS = lax.GatherDimensionNumbers(
    offset_dims=(), collapsed_slice_dims=(0,), start_index_map=(0,))


def _bcast(vec, j):
    idxs = jnp.full((16, 1), j, jnp.int32)
    return lax.gather(vec, idxs, _DNUMS, (1,),
                      mode=lax.GatherScatterMode.PROMISE_IN_BOUNDS)


def _sc_body(idx_hbm, val_hbm, emb_hbm, out_hbm,
             idx_v, val_v, out_v,
             rows0, rows1, rows2, rows3,
             sem0, sem1, sem2, sem3):
    rows = (rows0, rows1, rows2, rows3)
    sems = (sem0, sem1, sem2, sem3)
    wid = lax.axis_index("s") * 2 + lax.axis_index("c")

    pltpu.sync_copy(idx_hbm.at[pl.ds(wid * (BPW * HP), BPW * HP)], idx_v)
    pltpu.sync_copy(val_hbm.at[pl.ds(wid * (BPW * VW), BPW * VW)], val_v)

    def start(chunk, b):
        pltpu.make_async_copy(
            emb_hbm.at[idx_v.at[pl.ds(chunk * GSZ, GSZ)]], rows[b], sems[b]
        ).start()

    def wait(b):
        pltpu.make_async_copy(
            emb_hbm.at[idx_v.at[pl.ds(0, GSZ)]], rows[b], sems[b]
        ).wait()

    for b in range(NBUF):
        start(jnp.int32(b), b)

    def outer(i, carry):
        c0 = i * NBUF
        for b in range(NBUF):
            chunk = c0 + b
            wait(b)
            rbuf = rows[b]

            @pl.when(chunk + NBUF < NCHUNK)
            def _():
                start(chunk + NBUF, b)

            for r in range(CB):
                row = chunk * CB + r
                accs = [jnp.zeros((16,), jnp.float32) for _ in range(KD)]
                for g in range(4):
                    nj = 16 if g < 3 else HP - 48
                    wv = val_v[pl.ds(row * VW + g * 16, 16)]
                    for j in range(nj):
                        bw = _bcast(wv, j)
                        rr = r * HP + g * 16 + j
                        for k in range(KD):
                            accs[k] = accs[k] + bw * rbuf[rr, pl.ds(k * 16, 16)]
                for k in range(KD):
                    out_v[row, pl.ds(k * 16, 16)] = accs[k]
        return carry

    lax.fori_loop(0, NCHUNK // NBUF, outer, jnp.int32(0))
    pltpu.sync_copy(out_v, out_hbm.at[pl.ds(wid * BPW, BPW)])


_sc_call = functools.partial(
    pl.kernel,
    out_type=jax.ShapeDtypeStruct((B, D), jnp.float32),
    mesh=plsc.VectorSubcoreMesh(core_axis_name="c", subcore_axis_name="s"),
    scratch_types=[
        pltpu.VMEM((BPW * HP,), jnp.int32),
        pltpu.VMEM((BPW * VW,), jnp.float32),
        pltpu.VMEM((BPW, D), jnp.float32),
    ] + [pltpu.VMEM((GSZ, D), jnp.float32) for _ in range(NBUF)]
      + [pltpu.SemaphoreType.DMA for _ in range(NBUF)],
    compiler_params=pltpu.CompilerParams(use_tc_tiling_on_sc=False),
)(_sc_body)


def kernel(idx, val, embedding):
    npad = HP - HIST
    pad_rows = (jnp.arange(B * npad, dtype=jnp.int32) % VOCAB).reshape(B, npad)
    idxp = jnp.concatenate([idx.astype(jnp.int32), pad_rows], axis=1).reshape(-1)
    valp = jnp.pad(val, ((0, 0), (0, VW - HIST))).reshape(-1)
    out = _sc_call(idxp, valp, embedding)
    return out.reshape(B, 1, D)


# speedup vs baseline: 1.2195x; 1.2195x over previous
import functools

import jax
import jax.numpy as jnp
from jax import lax
from jax.experimental import pallas as pl
from jax.experimental.pallas import tpu as pltpu
from jax.experimental.pallas import tpu_sc as plsc

VOCAB = 100000
D = 64
B = 4096
HIST = 50
HP = 56
VW = 64
NW = 32
BPW = B // NW
CB = 2
NCHUNK = BPW // CB
GSZ = CB * VW
NBUF = 4
KD = D // 16

_DNUMS = lax.GatherDimensionNumbers(
    offset_dims=(), collapsed_slice_dims=(0,), start_index_map=(0,))


def _bcast(vec, j):
    idxs = jnp.full((16, 1), j, jnp.int32)
    return lax.gather(vec, idxs, _DNUMS, (1,),
                      mode=lax.GatherScatterMode.PROMISE_IN_BOUNDS)


def _sc_body(idx_hbm, val_hbm, emb_hbm, out_hbm,
             idx_v, val_v, out_v,
             rows0, rows1, rows2, rows3,
             sem0, sem1, sem2, sem3):
    rows = (rows0, rows1, rows2, rows3)
    sems = (sem0, sem1, sem2, sem3)
    wid = lax.axis_index("s") * 2 + lax.axis_index("c")

    pltpu.sync_copy(idx_hbm.at[pl.ds(wid * (BPW * VW), BPW * VW)], idx_v)
    pltpu.sync_copy(val_hbm.at[pl.ds(wid * (BPW * VW), BPW * VW)], val_v)

    def start(chunk, b):
        pltpu.make_async_copy(
            emb_hbm.at[idx_v.at[pl.ds(chunk * GSZ, GSZ)]], rows[b], sems[b]
        ).start()

    def wait(b):
        pltpu.make_async_copy(
            emb_hbm.at[idx_v.at[pl.ds(0, GSZ)]], rows[b], sems[b]
        ).wait()

    for b in range(NBUF):
        start(jnp.int32(b), b)

    def outer(i, carry):
        c0 = i * NBUF
        for b in range(NBUF):
            chunk = c0 + b
            wait(b)
            rbuf = rows[b]

            @pl.when(chunk + NBUF < NCHUNK)
            def _():
                start(chunk + NBUF, b)

            for r in range(CB):
                row = chunk * CB + r

                def gbody(g, acc, _r=r, _rbuf=rbuf, _row=row):
                    wv = val_v[pl.ds(_row * VW + g * 16, 16)]
                    accs = list(acc)
                    for j in range(16):
                        bw = _bcast(wv, j)
                        rr = _r * HP + g * 16 + j
                        for k in range(KD):
                            accs[k] = accs[k] + bw * _rbuf[rr, pl.ds(k * 16, 16)]
                    return tuple(accs)

                acc = lax.fori_loop(
                    0, 4, gbody,
                    tuple(jnp.zeros((16,), jnp.float32) for _ in range(KD)))
                for k in range(KD):
                    out_v[row, pl.ds(k * 16, 16)] = acc[k]
        return carry

    lax.fori_loop(0, NCHUNK // NBUF, outer, jnp.int32(0))
    pltpu.sync_copy(out_v, out_hbm.at[pl.ds(wid * BPW, BPW)])


_sc_call = functools.partial(
    pl.kernel,
    out_type=jax.ShapeDtypeStruct((B, D), jnp.float32),
    mesh=plsc.VectorSubcoreMesh(core_axis_name="c", subcore_axis_name="s"),
    scratch_types=[
        pltpu.VMEM((BPW * VW,), jnp.int32),
        pltpu.VMEM((BPW * VW,), jnp.float32),
        pltpu.VMEM((BPW, D), jnp.float32),
    ] + [pltpu.VMEM((GSZ, D), jnp.float32) for _ in range(NBUF)]
      + [pltpu.SemaphoreType.DMA for _ in range(NBUF)],
    compiler_params=pltpu.CompilerParams(use_tc_tiling_on_sc=False),
)(_sc_body)


def kernel(idx, val, embedding):
    npad = HP - HIST
    pad_rows = (jnp.arange(B * npad, dtype=jnp.int32) % VOCAB).reshape(B, npad)
    x = jnp.concatenate([idx.astype(jnp.int32), pad_rows], axis=1)
    x = x.reshape(B // CB, CB * HP)
    junk = (jnp.arange((B // CB) * (CB * VW - CB * HP), dtype=jnp.int32)
            % VOCAB).reshape(B // CB, CB * VW - CB * HP)
    idxp = jnp.concatenate([x, junk], axis=1).reshape(-1)
    valp = jnp.pad(val, ((0, 0), (0, VW - HIST))).reshape(-1)
    out = _sc_call(idxp, valp, embedding)
    return out.reshape(B, 1, D)


# speedup vs baseline: 1.2337x; 1.0116x over previous
import functools

import jax
import jax.numpy as jnp
from jax import lax
from jax.experimental import pallas as pl
from jax.experimental.pallas import tpu as pltpu
from jax.experimental.pallas import tpu_sc as plsc

VOCAB = 100000
D = 64
B = 4096
HIST = 50
HP = 56
VW = 64
NW = 32
BPW = B // NW
CB = 2
NCHUNK = BPW // CB
GSZ = CB * HP
NBUF = 4
KD = D // 16

_DNUMS = lax.GatherDimensionNumbers(
    offset_dims=(), collapsed_slice_dims=(0,), start_index_map=(0,))


def _bcast(vec, j):
    idxs = jnp.full((16, 1), j, jnp.int32)
    return lax.gather(vec, idxs, _DNUMS, (1,),
                      mode=lax.GatherScatterMode.PROMISE_IN_BOUNDS)


def _sc_body(idx_hbm, val_hbm, emb_hbm, out_hbm,
             idx_v, val_v, out_v,
             rows0, rows1, rows2, rows3,
             sem0, sem1, sem2, sem3):
    rows = (rows0, rows1, rows2, rows3)
    sems = (sem0, sem1, sem2, sem3)
    wid = lax.axis_index("s") * 2 + lax.axis_index("c")

    pltpu.sync_copy(idx_hbm.at[pl.ds(wid * (BPW * VW), BPW * VW)], idx_v)
    pltpu.sync_copy(val_hbm.at[pl.ds(wid * (BPW * VW), BPW * VW)], val_v)

    def start(chunk, b):
        pltpu.make_async_copy(
            emb_hbm.at[idx_v.at[pl.ds(chunk * (CB * VW), GSZ)]], rows[b], sems[b]
        ).start()

    def wait(b):
        pltpu.make_async_copy(
            emb_hbm.at[idx_v.at[pl.ds(0, GSZ)]], rows[b], sems[b]
        ).wait()

    for b in range(NBUF):
        start(jnp.int32(b), b)

    def outer(i, carry):
        c0 = i * NBUF
        for b in range(NBUF):
            chunk = c0 + b
            wait(b)
            rbuf = rows[b]

            @pl.when(chunk + NBUF < NCHUNK)
            def _():
                start(chunk + NBUF, b)

            for r in range(CB):
                row = chunk * CB + r

                def gbody(g, acc, _r=r, _rbuf=rbuf, _row=row):
                    wv = val_v[pl.ds(_row * VW + g * 16, 16)]
                    accs = list(acc)
                    for j in range(16):
                        bw = _bcast(wv, j)
                        rr = _r * HP + g * 16 + j
                        for k in range(KD):
                            accs[k] = accs[k] + bw * _rbuf[rr, pl.ds(k * 16, 16)]
                    return tuple(accs)

                acc = lax.fori_loop(
                    0, 4, gbody,
                    tuple(jnp.zeros((16,), jnp.float32) for _ in range(KD)))
                for k in range(KD):
                    out_v[row, pl.ds(k * 16, 16)] = acc[k]
        return carry

    lax.fori_loop(0, NCHUNK // NBUF, outer, jnp.int32(0))
    pltpu.sync_copy(out_v, out_hbm.at[pl.ds(wid * BPW, BPW)])


_sc_call = functools.partial(
    pl.kernel,
    out_type=jax.ShapeDtypeStruct((B, D), jnp.float32),
    mesh=plsc.VectorSubcoreMesh(core_axis_name="c", subcore_axis_name="s"),
    scratch_types=[
        pltpu.VMEM((BPW * VW,), jnp.int32),
        pltpu.VMEM((BPW * VW,), jnp.float32),
        pltpu.VMEM((BPW, D), jnp.float32),
    ] + [pltpu.VMEM((GSZ, D), jnp.float32) for _ in range(NBUF)]
      + [pltpu.SemaphoreType.DMA for _ in range(NBUF)],
    compiler_params=pltpu.CompilerParams(use_tc_tiling_on_sc=False),
)(_sc_body)


def kernel(idx, val, embedding):
    npad = HP - HIST
    pad_rows = (jnp.arange(B * npad, dtype=jnp.int32) % VOCAB).reshape(B, npad)
    x = jnp.concatenate([idx.astype(jnp.int32), pad_rows], axis=1)
    x = x.reshape(B // CB, CB * HP)
    junk = (jnp.arange((B // CB) * (CB * VW - CB * HP), dtype=jnp.int32)
            % VOCAB).reshape(B // CB, CB * VW - CB * HP)
    idxp = jnp.concatenate([x, junk], axis=1).reshape(-1)
    valp = jnp.pad(val, ((0, 0), (0, VW - HIST))).reshape(-1)
    out = _sc_call(idxp, valp, embedding)
    return out.reshape(B, 1, D)
